# baseline (speedup 1.0000x reference)
import functools

import jax
import jax.numpy as jnp
from jax import lax
from jax.experimental import pallas as pl
from jax.experimental.pallas import tpu as pltpu
from jax.experimental.pallas import tpu_sc as plsc
from jax._src.pallas import mpmd

DIM = 768
NUM_CORES = 2
NUM_SUBCORES = 16
NW = NUM_CORES * NUM_SUBCORES

CHUNK = 16


@functools.cache
def _make_gather(total_rows: int):
    b_per_w = total_rows // NW
    n_groups = b_per_w // (4 * CHUNK)
    vmesh = plsc.VectorSubcoreMesh(core_axis_name="c", subcore_axis_name="s")
    smesh = plsc.ScalarSubcoreMesh(axis_name="c")

    row_buf = pltpu.VMEM((CHUNK, DIM), jnp.float32) @ vmesh
    vdma = pltpu.SemaphoreType.DMA @ vmesh
    sdma = pltpu.SemaphoreType.DMA @ smesh
    vreg_sem = pltpu.SemaphoreType.REGULAR @ vmesh
    sreg_sem = pltpu.SemaphoreType.REGULAR @ smesh

    def tec_fn(idx_hbm, table_hbm, out_hbm, idx_v, rows_a0, rows_a1,
               rows_b0, rows_b1, spmem, ga0, ga1, gb0, gb1, sa0, sa1,
               xb0, xb1, dsem, ready0, ready1, ack0, ack1):
        del dsem
        cid = lax.axis_index("c")
        sid = lax.axis_index("s")
        wid = sid * NUM_CORES + cid
        base = wid * b_per_w
        rows_a = (rows_a0, rows_a1)
        rows_b = (rows_b0, rows_b1)
        gsem_a = (ga0, ga1)
        gsem_b = (gb0, gb1)
        ssem_a = (sa0, sa1)
        xsem_b = (xb0, xb1)
        ready = (ready0, ready1)
        ack = (ack0, ack1)

        pltpu.sync_copy(idx_hbm.at[pl.ds(base, b_per_w)], idx_v)

        def gather(chunk, buf, sem):
            return pltpu.async_copy(
                table_hbm.at[idx_v.at[pl.ds(chunk * CHUNK, CHUNK)]],
                buf, sem)

        def out_slice(chunk):
            return out_hbm.at[pl.ds(base + chunk * CHUNK, CHUNK)]

        def drain(src, sem):
            pltpu.make_async_copy(src, out_slice(0), sem).wait()

        def body(g, carry):
            c0 = 4 * g
            nonfirst = g > 0

            @pl.when(nonfirst)
            def _():
                drain(rows_a[0], ssem_a[0])
            gd0 = gather(c0, rows_a[0], gsem_a[0])
            gs0 = gather(c0 + 1, rows_b[0], gsem_b[0])
            gd0.wait()
            pltpu.async_copy(rows_a[0], out_slice(c0), ssem_a[0])
            @pl.when(nonfirst)
            def _():
                drain(rows_a[1], ssem_a[1])
            gd1 = gather(c0 + 2, rows_a[1], gsem_a[1])
            gs0.wait()

            @pl.when(nonfirst)
            def _():
                pl.semaphore_wait(ack[0], 1)
            x0 = pltpu.async_copy(rows_b[0], spmem.at[sid, 0], xsem_b[0])
            gs1 = gather(c0 + 3, rows_b[1], gsem_b[1])
            x0.wait()
            pl.semaphore_signal(ready[0], 1)
            gd1.wait()
            pltpu.async_copy(rows_a[1], out_slice(c0 + 2), ssem_a[1])
            gs1.wait()

            @pl.when(nonfirst)
            def _():
                pl.semaphore_wait(ack[1], 1)
            x1 = pltpu.async_copy(rows_b[1], spmem.at[sid, 1], xsem_b[1])
            x1.wait()
            pl.semaphore_signal(ready[1], 1)
            return carry

        lax.fori_loop(0, n_groups, body, 0)
        drain(rows_a[0], ssem_a[0])
        drain(rows_a[1], ssem_a[1])

    def scs_fn(idx_hbm, table_hbm, out_hbm, idx_v, rows_a0, rows_a1,
               rows_b0, rows_b1, spmem, ga0, ga1, gb0, gb1, sa0, sa1,
               xb0, xb1, dsem, ready0, ready1, ack0, ack1):
        del idx_hbm, table_hbm, idx_v, rows_a0, rows_a1, rows_b0, rows_b1
        del ga0, ga1, gb0, gb1, sa0, sa1, xb0, xb1
        cid = lax.axis_index("c")
        ready = (ready0, ready1)
        ack = (ack0, ack1)

        def body(g, carry):
            for par in (0, 1):
                chunk = 4 * g + 2 * par + 1
                pl.semaphore_wait(ready[par], NUM_SUBCORES)
                for t in range(NUM_SUBCORES):
                    base = (t * NUM_CORES + cid) * b_per_w
                    pltpu.async_copy(
                        spmem.at[t, par],
                        out_hbm.at[pl.ds(base + chunk * CHUNK, CHUNK)],
                        dsem)
                for t in range(NUM_SUBCORES):
                    base = (t * NUM_CORES + cid) * b_per_w
                    pltpu.make_async_copy(
                        spmem.at[t, par],
                        out_hbm.at[pl.ds(base + chunk * CHUNK, CHUNK)],
                        dsem).wait()

                @pl.when(g < n_groups - 1)
                def _():
                    for t in range(NUM_SUBCORES):
                        pl.semaphore_signal(
                            ack[par], 1, device_id={"s": t},
                            device_id_type=pl.DeviceIdType.MESH)
            return carry

        lax.fori_loop(0, n_groups, body, 0)

    return mpmd.mpmd_map(
        [(smesh, scs_fn), (vmesh, tec_fn)],
        out_types=jax.ShapeDtypeStruct((total_rows, DIM), jnp.float32),
        scratch_types=(
            pltpu.VMEM((b_per_w,), jnp.int32) @ vmesh,
            row_buf, row_buf, row_buf, row_buf,
            pltpu.VMEM_SHARED((NUM_SUBCORES, 2, CHUNK, DIM), jnp.float32),
            vdma, vdma, vdma, vdma, vdma, vdma, vdma, vdma,
            sdma,
            sreg_sem, sreg_sem,
            vreg_sem, vreg_sem,
        ),
    )


def kernel(inputs, table):
    batch, seqlen = inputs.shape
    flat_ids = inputs.reshape(-1).astype(jnp.int32)
    out = _make_gather(batch * seqlen)(flat_ids, table)
    return out.reshape(batch, seqlen, DIM)


# speedup vs baseline: 1.0024x; 1.0024x over previous
import functools

import jax
import jax.numpy as jnp
from jax import lax
from jax.experimental import pallas as pl
from jax.experimental.pallas import tpu as pltpu
from jax.experimental.pallas import tpu_sc as plsc
from jax._src.pallas import mpmd

DIM = 768
NUM_CORES = 2
NUM_SUBCORES = 16
CHUNK = 16
BLOCK = NUM_SUBCORES * CHUNK


@functools.cache
def _make_gather(total_rows: int):
    rows_per_sc = total_rows // NUM_CORES
    n_blocks = rows_per_sc // BLOCK
    n_groups = n_blocks // 4
    vmesh = plsc.VectorSubcoreMesh(core_axis_name="c", subcore_axis_name="s")
    smesh = plsc.ScalarSubcoreMesh(axis_name="c")

    row_buf = pltpu.VMEM((CHUNK, DIM), jnp.float32) @ vmesh
    vdma = pltpu.SemaphoreType.DMA @ vmesh
    sdma = pltpu.SemaphoreType.DMA @ smesh
    vreg_sem = pltpu.SemaphoreType.REGULAR @ vmesh
    sreg_sem = pltpu.SemaphoreType.REGULAR @ smesh

    def tec_fn(idx_hbm, table_hbm, out_hbm, idx_v, rows_a0, rows_a1,
               rows_b0, rows_b1, spmem, isem, ga0, ga1, gb0, gb1, sa0, sa1,
               xb0, xb1, dsem, ready0, ready1, ack0, ack1):
        del dsem
        cid = lax.axis_index("c")
        sid = lax.axis_index("s")
        sc_base = cid * rows_per_sc
        rows_a = (rows_a0, rows_a1)
        rows_b = (rows_b0, rows_b1)
        gsem_a = (ga0, ga1)
        gsem_b = (gb0, gb1)
        ssem_a = (sa0, sa1)
        xsem_b = (xb0, xb1)
        ready = (ready0, ready1)
        ack = (ack0, ack1)

        def stage(b, carry):
            pltpu.async_copy(
                idx_hbm.at[pl.ds(sc_base + b * BLOCK + sid * CHUNK, CHUNK)],
                idx_v.at[pl.ds(b * CHUNK, CHUNK)], isem)
            return carry
        lax.fori_loop(0, n_blocks, stage, 0)

        def drain_idx(b, carry):
            pltpu.make_async_copy(
                idx_hbm.at[pl.ds(sc_base, CHUNK)],
                idx_v.at[pl.ds(0, CHUNK)], isem).wait()
            return carry
        lax.fori_loop(0, n_blocks, drain_idx, 0)

        def gather(block, buf, sem):
            return pltpu.async_copy(
                table_hbm.at[idx_v.at[pl.ds(block * CHUNK, CHUNK)]],
                buf, sem)

        def out_slice(block):
            return out_hbm.at[
                pl.ds(sc_base + block * BLOCK + sid * CHUNK, CHUNK)]

        def drain(src, sem):
            pltpu.make_async_copy(src, out_slice(0), sem).wait()

        def body(g, carry):
            b0 = 4 * g
            nonfirst = g > 0

            @pl.when(nonfirst)
            def _():
                drain(rows_a[0], ssem_a[0])
            gd0 = gather(b0, rows_a[0], gsem_a[0])
            gs0 = gather(b0 + 1, rows_b[0], gsem_b[0])
            gd0.wait()
            pltpu.async_copy(rows_a[0], out_slice(b0), ssem_a[0])
            @pl.when(nonfirst)
            def _():
                drain(rows_a[1], ssem_a[1])
            gd1 = gather(b0 + 2, rows_a[1], gsem_a[1])
            gs0.wait()

            @pl.when(nonfirst)
            def _():
                pl.semaphore_wait(ack[0], 1)
            x0 = pltpu.async_copy(
                rows_b[0], spmem.at[0, pl.ds(sid * CHUNK, CHUNK)], xsem_b[0])
            gs1 = gather(b0 + 3, rows_b[1], gsem_b[1])
            x0.wait()
            pl.semaphore_signal(ready[0], 1)
            gd1.wait()
            pltpu.async_copy(rows_a[1], out_slice(b0 + 2), ssem_a[1])
            gs1.wait()

            @pl.when(nonfirst)
            def _():
                pl.semaphore_wait(ack[1], 1)
            x1 = pltpu.async_copy(
                rows_b[1], spmem.at[1, pl.ds(sid * CHUNK, CHUNK)], xsem_b[1])
            x1.wait()
            pl.semaphore_signal(ready[1], 1)
            return carry

        lax.fori_loop(0, n_groups, body, 0)
        drain(rows_a[0], ssem_a[0])
        drain(rows_a[1], ssem_a[1])

    def scs_fn(idx_hbm, table_hbm, out_hbm, idx_v, rows_a0, rows_a1,
               rows_b0, rows_b1, spmem, isem, ga0, ga1, gb0, gb1, sa0, sa1,
               xb0, xb1, dsem, ready0, ready1, ack0, ack1):
        del idx_hbm, table_hbm, idx_v, rows_a0, rows_a1, rows_b0, rows_b1
        del isem, ga0, ga1, gb0, gb1, sa0, sa1, xb0, xb1
        cid = lax.axis_index("c")
        sc_base = cid * rows_per_sc
        ready = (ready0, ready1)
        ack = (ack0, ack1)

        def body(g, carry):
            for par in (0, 1):
                block = 4 * g + 2 * par + 1
                pl.semaphore_wait(ready[par], NUM_SUBCORES)
                pltpu.async_copy(
                    spmem.at[par],
                    out_hbm.at[pl.ds(sc_base + block * BLOCK, BLOCK)],
                    dsem)
                pltpu.make_async_copy(
                    spmem.at[par],
                    out_hbm.at[pl.ds(sc_base, BLOCK)],
                    dsem).wait()

                @pl.when(g < n_groups - 1)
                def _():
                    for t in range(NUM_SUBCORES):
                        pl.semaphore_signal(
                            ack[par], 1, device_id={"s": t},
                            device_id_type=pl.DeviceIdType.MESH)
            return carry

        lax.fori_loop(0, n_groups, body, 0)

    return mpmd.mpmd_map(
        [(smesh, scs_fn), (vmesh, tec_fn)],
        out_types=jax.ShapeDtypeStruct((total_rows, DIM), jnp.float32),
        scratch_types=(
            pltpu.VMEM((n_blocks * CHUNK,), jnp.int32) @ vmesh,
            row_buf, row_buf, row_buf, row_buf,
            pltpu.VMEM_SHARED((2, BLOCK, DIM), jnp.float32),
            vdma,
            vdma, vdma, vdma, vdma, vdma, vdma, vdma, vdma,
            sdma,
            sreg_sem, sreg_sem,
            vreg_sem, vreg_sem,
        ),
    )


def kernel(inputs, table):
    batch, seqlen = inputs.shape
    flat_ids = inputs.reshape(-1).astype(jnp.int32)
    out = _make_gather(batch * seqlen)(flat_ids, table)
    return out.reshape(batch, seqlen, DIM)


# speedup vs baseline: 1.0258x; 1.0233x over previous
import functools

import jax
import jax.numpy as jnp
from jax import lax
from jax.experimental import pallas as pl
from jax.experimental.pallas import tpu as pltpu
from jax.experimental.pallas import tpu_sc as plsc

DIM = 768
NUM_CORES = 2
NUM_SUBCORES = 16
NW = NUM_CORES * NUM_SUBCORES

NBUF = 4
CHUNK = 32


@functools.cache
def _make_gather(total_rows: int):
    b_per_w = total_rows // NW
    n_chunks = b_per_w // CHUNK
    mesh = plsc.VectorSubcoreMesh(core_axis_name="c", subcore_axis_name="s")

    row_buf = pltpu.VMEM((CHUNK, DIM), jnp.float32)
    dma = pltpu.SemaphoreType.DMA

    @functools.partial(
        pl.kernel,
        mesh=mesh,
        out_type=jax.ShapeDtypeStruct((total_rows, DIM), jnp.float32),
        scratch_types=[
            pltpu.VMEM((b_per_w,), jnp.int32),
            row_buf, row_buf, row_buf, row_buf,
            dma, dma, dma, dma, dma, dma, dma, dma,
        ],
    )
    def gather_kernel(idx_hbm, table_hbm, out_hbm, idx_v, r0, r1, r2, r3,
                      g0, g1, g2, g3, s0, s1, s2, s3):
        wid = lax.axis_index("s") * NUM_CORES + lax.axis_index("c")
        base = wid * b_per_w
        rows = (r0, r1, r2, r3)
        gsem = (g0, g1, g2, g3)
        ssem = (s0, s1, s2, s3)

        pltpu.sync_copy(idx_hbm.at[pl.ds(base, b_per_w)], idx_v)

        def start_gather(i):
            buf = i % NBUF
            return pltpu.async_copy(
                table_hbm.at[idx_v.at[pl.ds(i * CHUNK, CHUNK)]],
                rows[buf], gsem[buf])

        def start_store(i):
            buf = i % NBUF
            return pltpu.async_copy(
                rows[buf], out_hbm.at[pl.ds(base + i * CHUNK, CHUNK)],
                ssem[buf])

        gathers = [None] * n_chunks
        stores = [None] * n_chunks
        for i in range(NBUF - 1):
            gathers[i] = start_gather(i)
        for i in range(n_chunks):
            if i + NBUF - 1 < n_chunks:
                if i >= 1:
                    stores[i - 1].wait()
                gathers[i + NBUF - 1] = start_gather(i + NBUF - 1)
            gathers[i].wait()
            stores[i] = start_store(i)
        for i in range(n_chunks - NBUF, n_chunks):
            stores[i].wait()

    return gather_kernel


def kernel(inputs, table):
    batch, seqlen = inputs.shape
    flat_ids = inputs.reshape(-1).astype(jnp.int32)
    out = _make_gather(batch * seqlen)(flat_ids, table)
    return out.reshape(batch, seqlen, DIM)
